# baseline (speedup 1.0000x reference)
import functools

import jax
import jax.numpy as jnp
from jax import lax
from jax.experimental import pallas as pl
from jax.experimental.pallas import tpu as pltpu
from jax.experimental.pallas import tpu_sc as plsc

BUCKETS = 1000000
HASH_C = 1315423911
C_HI = HASH_C >> 15
C_LO = HASH_C & 0x7FFF

BATCH = 4096
SEQ = 200
DIM = 64
MODEL_DIM = 128
TOKENS = BATCH * SEQ

NC, NS = 2, 16
NW = NC * NS
CHUNK = 128
G = TOKENS // (NW * CHUNK)


def _hash_body(ids_ref, out_ref):
    x = ids_ref[...]
    prev = pltpu.roll(x, jnp.int32(1), 1)
    i32 = lambda v: jnp.int32(v)
    a = prev * i32(C_HI)
    b = prev * i32(C_LO)
    a_hi = lax.shift_right_logical(a, i32(2))
    s = (a & i32(3)) * i32(32768) + b
    hi = a_hi + lax.shift_right_arithmetic(s, i32(17))
    lo = (s & i32(0x1FFFF)) ^ x
    h = lax.rem(hi, i32(BUCKETS))
    h1 = lax.shift_right_arithmetic(h, i32(10))
    h0 = h & i32(1023)
    t = h1 * i32(217728) + h0 * i32(131072) + lo
    pair = lax.rem(t, i32(BUCKETS))
    col = lax.broadcasted_iota(jnp.int32, x.shape, 1)
    out_ref[...] = jnp.where(col == 0, x, pair)


def _bigram_hash(ids32):
    bb = 512
    return pl.pallas_call(
        _hash_body,
        grid=(BATCH // bb,),
        in_specs=[pl.BlockSpec((bb, SEQ), lambda i: (i, jnp.int32(0)))],
        out_specs=pl.BlockSpec((bb, SEQ), lambda i: (i, jnp.int32(0))),
        out_shape=jax.ShapeDtypeStruct((BATCH, SEQ), jnp.int32),
    )(ids32)


_SC_MESH = plsc.VectorSubcoreMesh(core_axis_name="c", subcore_axis_name="s")


@functools.partial(
    pl.kernel,
    mesh=_SC_MESH,
    out_type=jax.ShapeDtypeStruct((NW, G, CHUNK, DIM), jnp.float32),
    scratch_types=[
        pltpu.VMEM((G, CHUNK), jnp.int32),
        pltpu.VMEM((CHUNK, DIM), jnp.float32),
        pltpu.SemaphoreType.DMA,
    ],
    compiler_params=pltpu.CompilerParams(use_tc_tiling_on_sc=False),
)
def _sc_gather(idx_hbm, table_hbm, out_hbm, idx_v, rows_v, sem):
    wid = lax.axis_index("s") * NC + lax.axis_index("c")
    pltpu.sync_copy(idx_hbm.at[wid], idx_v)

    def body(g, carry):
        pltpu.async_copy(table_hbm.at[idx_v.at[g]], rows_v, sem).wait()
        pltpu.sync_copy(rows_v, out_hbm.at[wid, g])
        return carry

    lax.fori_loop(0, G, body, jnp.int32(0))


def _mm_body(x_ref, w_ref, s_ref, o_ref):
    acc = lax.dot_general(
        x_ref[...], w_ref[...],
        (((1,), (1,)), ((), ())),
        preferred_element_type=jnp.float32,
    )
    o_ref[...] = s_ref[0] * acc


def _project(rows, w, scale1):
    bm = 1024
    return pl.pallas_call(
        _mm_body,
        grid=(TOKENS // bm,),
        in_specs=[
            pl.BlockSpec((bm, DIM), lambda i: (i, jnp.int32(0))),
            pl.BlockSpec((MODEL_DIM, DIM), lambda i: (jnp.int32(0), jnp.int32(0))),
            pl.BlockSpec((1,), lambda i: (jnp.int32(0),), memory_space=pltpu.SMEM),
        ],
        out_specs=pl.BlockSpec((bm, MODEL_DIM), lambda i: (i, jnp.int32(0))),
        out_shape=jax.ShapeDtypeStruct((TOKENS, MODEL_DIM), jnp.float32),
    )(rows, w, scale1)


def kernel(input_ids, embed, W_proj, scale):
    ids32 = input_ids.astype(jnp.int32)
    pair = _bigram_hash(ids32)
    idx = pair.reshape(NW, G, CHUNK)
    rows = _sc_gather(idx, embed).reshape(TOKENS, DIM)
    out = _project(rows, W_proj, scale.astype(jnp.float32).reshape(1))
    return out.reshape(BATCH, SEQ, MODEL_DIM)


# speedup vs baseline: 1.0025x; 1.0025x over previous
import functools

import jax
import jax.numpy as jnp
from jax import lax
from jax.experimental import pallas as pl
from jax.experimental.pallas import tpu as pltpu
from jax.experimental.pallas import tpu_sc as plsc

BUCKETS = 1000000
HASH_C = 1315423911
C_HI = HASH_C >> 15
C_LO = HASH_C & 0x7FFF

BATCH = 4096
SEQ = 200
DIM = 64
MODEL_DIM = 128
TOKENS = BATCH * SEQ

NC, NS = 2, 16
NW = NC * NS
CHUNK = 128
G = TOKENS // (NW * CHUNK)


def _hash_body(ids_ref, out_ref):
    x = ids_ref[...]
    prev = pltpu.roll(x, jnp.int32(1), 1)
    i32 = lambda v: jnp.int32(v)
    a = prev * i32(C_HI)
    b = prev * i32(C_LO)
    a_hi = lax.shift_right_logical(a, i32(2))
    s = (a & i32(3)) * i32(32768) + b
    hi = a_hi + lax.shift_right_arithmetic(s, i32(17))
    lo = (s & i32(0x1FFFF)) ^ x
    h = lax.rem(hi, i32(BUCKETS))
    h1 = lax.shift_right_arithmetic(h, i32(10))
    h0 = h & i32(1023)
    t = h1 * i32(217728) + h0 * i32(131072) + lo
    pair = lax.rem(t, i32(BUCKETS))
    col = lax.broadcasted_iota(jnp.int32, x.shape, 1)
    out_ref[...] = jnp.where(col == 0, x, pair)


def _bigram_hash(ids32):
    bb = 512
    return pl.pallas_call(
        _hash_body,
        grid=(BATCH // bb,),
        in_specs=[pl.BlockSpec((bb, SEQ), lambda i: (i, jnp.int32(0)))],
        out_specs=pl.BlockSpec((bb, SEQ), lambda i: (i, jnp.int32(0))),
        out_shape=jax.ShapeDtypeStruct((BATCH, SEQ), jnp.int32),
    )(ids32)


_SC_MESH = plsc.VectorSubcoreMesh(core_axis_name="c", subcore_axis_name="s")


@functools.partial(
    pl.kernel,
    mesh=_SC_MESH,
    out_type=jax.ShapeDtypeStruct((TOKENS, DIM), jnp.float32),
    scratch_types=[
        pltpu.VMEM((G, CHUNK), jnp.int32),
        pltpu.VMEM((CHUNK, DIM), jnp.float32),
        pltpu.SemaphoreType.DMA,
    ],
    compiler_params=pltpu.CompilerParams(use_tc_tiling_on_sc=False),
)
def _sc_gather(idx_hbm, table_hbm, out_hbm, idx_v, rows_v, sem):
    wid = lax.axis_index("s") * NC + lax.axis_index("c")
    pltpu.sync_copy(idx_hbm.at[wid], idx_v)

    base = lax.mul(wid, jnp.int32(G * CHUNK))

    def body(g, carry):
        off = lax.add(base, lax.mul(g, jnp.int32(CHUNK)))
        pltpu.async_copy(table_hbm.at[idx_v.at[g]], rows_v, sem).wait()
        pltpu.sync_copy(rows_v, out_hbm.at[pl.ds(off, CHUNK)])
        return carry

    lax.fori_loop(jnp.int32(0), jnp.int32(G), body, jnp.int32(0))


def _mm_body(x_ref, w_ref, s_ref, o_ref):
    acc = lax.dot_general(
        x_ref[...], w_ref[...],
        (((1,), (1,)), ((), ())),
        preferred_element_type=jnp.float32,
    )
    o_ref[...] = s_ref[0] * acc


def _project(rows, w, scale1):
    bm = 1024
    return pl.pallas_call(
        _mm_body,
        grid=(TOKENS // bm,),
        in_specs=[
            pl.BlockSpec((bm, DIM), lambda i: (i, jnp.int32(0))),
            pl.BlockSpec((MODEL_DIM, DIM), lambda i: (jnp.int32(0), jnp.int32(0))),
            pl.BlockSpec((1,), lambda i: (jnp.int32(0),), memory_space=pltpu.SMEM),
        ],
        out_specs=pl.BlockSpec((bm, MODEL_DIM), lambda i: (i, jnp.int32(0))),
        out_shape=jax.ShapeDtypeStruct((TOKENS, MODEL_DIM), jnp.float32),
    )(rows, w, scale1)


def kernel(input_ids, embed, W_proj, scale):
    ids32 = input_ids.astype(jnp.int32)
    pair = _bigram_hash(ids32)
    idx = pair.reshape(NW, G, CHUNK)
    rows = _sc_gather(idx, embed)
    out = _project(rows, W_proj, scale.astype(jnp.float32).reshape(1))
    return out.reshape(BATCH, SEQ, MODEL_DIM)


# speedup vs baseline: 1.1342x; 1.1314x over previous
import functools

import jax
import jax.numpy as jnp
from jax import lax
from jax.experimental import pallas as pl
from jax.experimental.pallas import tpu as pltpu
from jax.experimental.pallas import tpu_sc as plsc

BUCKETS = 1000000
HASH_C = 1315423911
C_HI = HASH_C >> 15
C_LO = HASH_C & 0x7FFF

BATCH = 4096
SEQ = 200
DIM = 64
MODEL_DIM = 128
TOKENS = BATCH * SEQ

NC, NS = 2, 16
NW = NC * NS
CHUNK = 128
G = TOKENS // (NW * CHUNK)


def _hash_body(ids_ref, out_ref):
    x = ids_ref[...]
    prev = pltpu.roll(x, jnp.int32(1), 1)
    i32 = lambda v: jnp.int32(v)
    a = prev * i32(C_HI)
    b = prev * i32(C_LO)
    a_hi = lax.shift_right_logical(a, i32(2))
    s = (a & i32(3)) * i32(32768) + b
    hi = a_hi + lax.shift_right_arithmetic(s, i32(17))
    lo = (s & i32(0x1FFFF)) ^ x
    h = lax.rem(hi, i32(BUCKETS))
    h1 = lax.shift_right_arithmetic(h, i32(10))
    h0 = h & i32(1023)
    t = h1 * i32(217728) + h0 * i32(131072) + lo
    pair = lax.rem(t, i32(BUCKETS))
    col = lax.broadcasted_iota(jnp.int32, x.shape, 1)
    out_ref[...] = jnp.where(col == 0, x, pair)


def _bigram_hash(ids32):
    bb = 512
    return pl.pallas_call(
        _hash_body,
        grid=(BATCH // bb,),
        in_specs=[pl.BlockSpec((bb, SEQ), lambda i: (i, jnp.int32(0)))],
        out_specs=pl.BlockSpec((bb, SEQ), lambda i: (i, jnp.int32(0))),
        out_shape=jax.ShapeDtypeStruct((BATCH, SEQ), jnp.int32),
    )(ids32)


_SC_MESH = plsc.VectorSubcoreMesh(core_axis_name="c", subcore_axis_name="s")


@functools.partial(
    pl.kernel,
    mesh=_SC_MESH,
    out_type=jax.ShapeDtypeStruct((TOKENS, DIM), jnp.float32),
    scratch_types=[
        pltpu.VMEM((G, CHUNK), jnp.int32),
        pltpu.VMEM((2, CHUNK, DIM), jnp.float32),
        pltpu.SemaphoreType.DMA,
        pltpu.SemaphoreType.DMA,
        pltpu.SemaphoreType.DMA,
    ],
    compiler_params=pltpu.CompilerParams(use_tc_tiling_on_sc=False),
)
def _sc_gather(idx_hbm, table_hbm, out_hbm, idx_v, rows_v, gsem, wsem0, wsem1):
    wid = lax.axis_index("s") * NC + lax.axis_index("c")
    pltpu.sync_copy(idx_hbm.at[wid], idx_v)

    base = lax.mul(wid, jnp.int32(G * CHUNK))
    wsems = (wsem0, wsem1)

    def body(g, carry):
        off = lax.add(base, lax.mul(g, jnp.int32(CHUNK)))
        for b in range(2):

            @pl.when(lax.rem(g, jnp.int32(2)) == b)
            def _():
                @pl.when(g >= 2)
                def _():
                    pltpu.make_async_copy(
                        rows_v.at[jnp.int32(b)], out_hbm.at[pl.ds(off, CHUNK)], wsems[b]
                    ).wait()

                pltpu.async_copy(table_hbm.at[idx_v.at[g]], rows_v.at[jnp.int32(b)], gsem).wait()
                pltpu.async_copy(rows_v.at[jnp.int32(b)], out_hbm.at[pl.ds(off, CHUNK)], wsems[b])

        return carry

    lax.fori_loop(jnp.int32(0), jnp.int32(G), body, jnp.int32(0))
    for b in range(2):
        pltpu.make_async_copy(
            rows_v.at[jnp.int32(b)], out_hbm.at[pl.ds(base, CHUNK)], wsems[b]
        ).wait()


BM2 = 1024


def _mm_body(x_ref, w_ref, s_ref, o_ref):
    x2 = x_ref[...].reshape(BM2, 128)
    s = s_ref[0]
    xa = x2[:, :DIM]
    xb = x2[:, DIM:]
    dn = (((1,), (1,)), ((), ()))
    oa = lax.dot_general(xa, w_ref[...], dn, preferred_element_type=jnp.float32)
    ob = lax.dot_general(xb, w_ref[...], dn, preferred_element_type=jnp.float32)
    o_ref[:, :MODEL_DIM] = s * oa
    o_ref[:, MODEL_DIM:] = s * ob


def _project(rows_flat, w, scale1):
    return pl.pallas_call(
        _mm_body,
        grid=(TOKENS // (2 * BM2),),
        in_specs=[
            pl.BlockSpec((BM2 * 128,), lambda i: (i,)),
            pl.BlockSpec((MODEL_DIM, DIM), lambda i: (jnp.int32(0), jnp.int32(0))),
            pl.BlockSpec((1,), lambda i: (jnp.int32(0),), memory_space=pltpu.SMEM),
        ],
        out_specs=pl.BlockSpec((BM2, 2 * MODEL_DIM), lambda i: (i, jnp.int32(0))),
        out_shape=jax.ShapeDtypeStruct((TOKENS // 2, 2 * MODEL_DIM), jnp.float32),
    )(rows_flat, w, scale1)


def kernel(input_ids, embed, W_proj, scale):
    ids32 = input_ids.astype(jnp.int32)
    pair = _bigram_hash(ids32)
    idx = pair.reshape(NW, G, CHUNK)
    rows = _sc_gather(idx, embed)
    rows_flat = rows.reshape(TOKENS * DIM)
    out = _project(rows_flat, W_proj, scale.astype(jnp.float32).reshape(1))
    return out.reshape(BATCH, SEQ, MODEL_DIM)


# speedup vs baseline: 2.0430x; 1.8012x over previous
import functools

import jax
import jax.numpy as jnp
from jax import lax
from jax.experimental import pallas as pl
from jax.experimental.pallas import tpu as pltpu
from jax.experimental.pallas import tpu_sc as plsc

BUCKETS = 1000000
HASH_C = 1315423911
C_HI = HASH_C >> 15
C_LO = HASH_C & 0x7FFF

BATCH = 4096
SEQ = 200
DIM = 64
MODEL_DIM = 128
TOKENS = BATCH * SEQ

NC, NS = 2, 16
NW = NC * NS
CHUNK = 128
G = TOKENS // (NW * CHUNK)


def _hash_body(ids_ref, out_ref):
    x = ids_ref[...]
    prev = pltpu.roll(x, jnp.int32(1), 1)
    i32 = lambda v: jnp.int32(v)
    a = prev * i32(C_HI)
    b = prev * i32(C_LO)
    a_hi = lax.shift_right_logical(a, i32(2))
    s = (a & i32(3)) * i32(32768) + b
    hi = a_hi + lax.shift_right_arithmetic(s, i32(17))
    lo = (s & i32(0x1FFFF)) ^ x
    h = lax.rem(hi, i32(BUCKETS))
    h1 = lax.shift_right_arithmetic(h, i32(10))
    h0 = h & i32(1023)
    t = h1 * i32(217728) + h0 * i32(131072) + lo
    pair = lax.rem(t, i32(BUCKETS))
    col = lax.broadcasted_iota(jnp.int32, x.shape, 1)
    out_ref[...] = jnp.where(col == 0, x, pair)


def _bigram_hash(ids32):
    bb = 512
    return pl.pallas_call(
        _hash_body,
        grid=(BATCH // bb,),
        in_specs=[pl.BlockSpec((bb, SEQ), lambda i: (i, jnp.int32(0)))],
        out_specs=pl.BlockSpec((bb, SEQ), lambda i: (i, jnp.int32(0))),
        out_shape=jax.ShapeDtypeStruct((BATCH, SEQ), jnp.int32),
    )(ids32)


BK = 2048


def _proj_body(et_ref, w_ref, s_ref, o_ref):
    acc = lax.dot_general(
        et_ref[...], w_ref[...],
        (((0,), (1,)), ((), ())),
        preferred_element_type=jnp.float32,
    )
    o_ref[...] = s_ref[0] * acc


def _project_table(embed_t, w, scale1):
    return pl.pallas_call(
        _proj_body,
        grid=((BUCKETS + BK - 1) // BK,),
        in_specs=[
            pl.BlockSpec((DIM, BK), lambda i: (jnp.int32(0), i)),
            pl.BlockSpec((MODEL_DIM, DIM), lambda i: (jnp.int32(0), jnp.int32(0))),
            pl.BlockSpec((1,), lambda i: (jnp.int32(0),), memory_space=pltpu.SMEM),
        ],
        out_specs=pl.BlockSpec((BK, MODEL_DIM), lambda i: (i, jnp.int32(0))),
        out_shape=jax.ShapeDtypeStruct((BUCKETS, MODEL_DIM), jnp.float32),
    )(embed_t, w, scale1)


_SC_MESH = plsc.VectorSubcoreMesh(core_axis_name="c", subcore_axis_name="s")


@functools.partial(
    pl.kernel,
    mesh=_SC_MESH,
    out_type=jax.ShapeDtypeStruct((TOKENS, MODEL_DIM), jnp.float32),
    scratch_types=[
        pltpu.VMEM((G, CHUNK), jnp.int32),
        pltpu.VMEM((2, CHUNK, MODEL_DIM), jnp.float32),
        pltpu.SemaphoreType.DMA,
        pltpu.SemaphoreType.DMA,
        pltpu.SemaphoreType.DMA,
    ],
    compiler_params=pltpu.CompilerParams(use_tc_tiling_on_sc=False),
)
def _sc_gather(idx_hbm, table_hbm, out_hbm, idx_v, rows_v, gsem, wsem0, wsem1):
    wid = lax.axis_index("s") * NC + lax.axis_index("c")
    pltpu.sync_copy(idx_hbm.at[wid], idx_v)

    base = lax.mul(wid, jnp.int32(G * CHUNK))
    wsems = (wsem0, wsem1)

    def body(g, carry):
        off = lax.add(base, lax.mul(g, jnp.int32(CHUNK)))
        for b in range(2):

            @pl.when(lax.rem(g, jnp.int32(2)) == b)
            def _():
                @pl.when(g >= 2)
                def _():
                    pltpu.make_async_copy(
                        rows_v.at[jnp.int32(b)], out_hbm.at[pl.ds(off, CHUNK)], wsems[b]
                    ).wait()

                pltpu.async_copy(table_hbm.at[idx_v.at[g]], rows_v.at[jnp.int32(b)], gsem).wait()
                pltpu.async_copy(rows_v.at[jnp.int32(b)], out_hbm.at[pl.ds(off, CHUNK)], wsems[b])

        return carry

    lax.fori_loop(jnp.int32(0), jnp.int32(G), body, jnp.int32(0))
    for b in range(2):
        pltpu.make_async_copy(
            rows_v.at[jnp.int32(b)], out_hbm.at[pl.ds(base, CHUNK)], wsems[b]
        ).wait()


def kernel(input_ids, embed, W_proj, scale):
    ids32 = input_ids.astype(jnp.int32)
    pair = _bigram_hash(ids32)
    idx = pair.reshape(NW, G, CHUNK)
    embed_t = embed.T
    table = _project_table(embed_t, W_proj, scale.astype(jnp.float32).reshape(1))
    out = _sc_gather(idx, table)
    return out.reshape(BATCH, SEQ, MODEL_DIM)
